# baseline (speedup 1.0000x reference)
import functools

import jax
import jax.numpy as jnp
from jax import lax
from jax.experimental import pallas as pl
from jax.experimental.pallas import tpu as pltpu
from jax.experimental.pallas import tpu_sc as plsc

_N = 10000
_E = 320000
_D = 128
_DE = 16
_H = 128

_NC = 2
_NS = 16
_NW = _NC * _NS
_C = 80
_PER_W = _E // _NW
_NCH = _PER_W // _C
_HW = _H // 2
_NP = 10240
_RPS = _NP // _NS
_ZR = 64


def _pq_body(x_ref, wa_ref, wb_ref, p_ref, q_ref):
    x = x_ref[...]
    p_ref[...] = jnp.dot(x, wa_ref[0], preferred_element_type=jnp.float32)
    q_ref[...] = jnp.dot(x, wb_ref[0], preferred_element_type=jnp.float32)


def _node_tables(x2, wa2, wb2):
    bs = 2000
    nb = _N // bs
    return pl.pallas_call(
        _pq_body,
        grid=(2 * nb,),
        in_specs=[
            pl.BlockSpec((bs, _D), lambda i: (i, 0)),
            pl.BlockSpec((1, _D, _H), lambda i: (i // nb, 0, 0)),
            pl.BlockSpec((1, _D, _H), lambda i: (i // nb, 0, 0)),
        ],
        out_specs=[
            pl.BlockSpec((bs, _H), lambda i: (i, 0)),
            pl.BlockSpec((bs, _H), lambda i: (i, 0)),
        ],
        out_shape=[
            jax.ShapeDtypeStruct((2 * _N, _H), jnp.float32),
            jax.ShapeDtypeStruct((2 * _N, _H), jnp.float32),
        ],
    )(x2, wa2, wb2)


def _edge_body(g_ref, e_ref, w1c_ref, b1_ref, w2_ref, b2_ref, h3_ref):
    pre = (g_ref[...]
           + jnp.dot(e_ref[...], w1c_ref[0],
                     preferred_element_type=jnp.float32) + b1_ref[0])
    h = jax.nn.silu(pre)
    h2 = jax.nn.silu(jnp.dot(h, w2_ref[0],
                             preferred_element_type=jnp.float32) + b2_ref[0])
    h3_ref[...] = jax.nn.silu(h2)


def _edge_mlp(g2, e2, w1c2, b12, w22, b22):
    bs = 4000
    nb = _E // bs
    return pl.pallas_call(
        _edge_body,
        grid=(2 * nb,),
        in_specs=[
            pl.BlockSpec((bs, _H), lambda i: (i, 0)),
            pl.BlockSpec((bs, _DE), lambda i: (i, 0)),
            pl.BlockSpec((1, _DE, _H), lambda i: (i // nb, 0, 0)),
            pl.BlockSpec((1, 1, _H), lambda i: (i // nb, 0, 0)),
            pl.BlockSpec((1, _H, _H), lambda i: (i // nb, 0, 0)),
            pl.BlockSpec((1, 1, _H), lambda i: (i // nb, 0, 0)),
        ],
        out_specs=pl.BlockSpec((bs, _H), lambda i: (i, 0)),
        out_shape=jax.ShapeDtypeStruct((_E2, _H), jnp.float32),
    )(g2, e2, w1c2, b12, w22, b22)


def _final_body(agg_ref, wu_ref, bu_ref, out_ref):
    a = agg_ref[0] + agg_ref[1]
    out_ref[...] = jnp.dot(a, wu_ref[...],
                           preferred_element_type=jnp.float32) + bu_ref[...]


def _final_update(agg, wu, bu):
    bs = 2000
    return pl.pallas_call(
        _final_body,
        grid=(_N // bs,),
        in_specs=[
            pl.BlockSpec((2, bs, _H), lambda i: (0, i, 0)),
            pl.BlockSpec((_H, _H), lambda i: (0, 0)),
            pl.BlockSpec((1, _H), lambda i: (0, 0)),
        ],
        out_specs=pl.BlockSpec((bs, _H), lambda i: (i, 0)),
        out_shape=jax.ShapeDtypeStruct((_N, _H), jnp.float32),
    )(agg, wu, bu.reshape(1, _H))


def _cell_body(x_ref, wc_ref, bc_ref, out_ref):
    out_ref[...] = jax.nn.silu(
        jnp.dot(x_ref[...], wc_ref[...], preferred_element_type=jnp.float32)
        + bc_ref[...])


def _cell_branch(x, wc, bc):
    bs = 2000
    return pl.pallas_call(
        _cell_body,
        grid=(_N // bs,),
        in_specs=[
            pl.BlockSpec((bs, _D), lambda i: (i, 0)),
            pl.BlockSpec((_D, _H), lambda i: (0, 0)),
            pl.BlockSpec((1, _H), lambda i: (0, 0)),
        ],
        out_specs=pl.BlockSpec((bs, _H), lambda i: (i, 0)),
        out_shape=jax.ShapeDtypeStruct((_N, _H), jnp.float32),
    )(x, wc, bc.reshape(1, _H))


_MESH = plsc.VectorSubcoreMesh(core_axis_name="c", subcore_axis_name="s")


_NSLOT = 3
_E2 = 2 * _E
_PER_W2 = _E2 // _NW
_NCH2 = _PER_W2 // _C


def _sc_gather_body(p_hbm, q_hbm, dst3_hbm, src3_hbm, g_hbm,
                    idx_d, idx_s, *rest):
    wid = lax.axis_index("s") * _NC + lax.axis_index("c")
    base = wid * _PER_W2

    pltpu.sync_copy(dst3_hbm.at[wid], idx_d)
    pltpu.sync_copy(src3_hbm.at[wid], idx_s)

    rows = rest[:2 * _NSLOT]
    sems = rest[2 * _NSLOT:]
    bufs = tuple(
        (rows[2 * b], rows[2 * b + 1],
         sems[3 * b], sems[3 * b + 1], sems[3 * b + 2])
        for b in range(_NSLOT))

    def start(c, b, first=False):
        prow, qrow, sp, sq, st = bufs[b]
        if not first:
            pltpu.make_async_copy(prow, g_hbm.at[pl.ds(base, _C)], st).wait()
        pltpu.async_copy(p_hbm.at[idx_d.at[c]], prow, sp)
        pltpu.async_copy(q_hbm.at[idx_s.at[c]], qrow, sq)

    def finish(c, b, last=False):
        prow, qrow, sp, sq, st = bufs[b]
        pltpu.make_async_copy(p_hbm.at[idx_d.at[c]], prow, sp).wait()
        pltpu.make_async_copy(q_hbm.at[idx_s.at[c]], qrow, sq).wait()

        def add_row(r, c2):
            for j in range(_H // 16):
                sl = pl.ds(j * 16, 16)
                plsc.addupdate(prow.at[r, sl], qrow[r, sl])
            return c2

        lax.fori_loop(0, _C, add_row, 0)
        dsub = pl.ds(base + c * _C, _C)
        pltpu.async_copy(prow, g_hbm.at[dsub], st)
        if last:
            pltpu.make_async_copy(prow, g_hbm.at[dsub], st).wait()

    for b in range(_NSLOT):
        start(b, b, first=True)

    def group(k, carry):
        c0 = _NSLOT * k
        for j in range(_NSLOT):
            finish(c0 + j, j)
            start(c0 + _NSLOT + j, j)
        return carry

    nf = _NCH2 // _NSLOT - 1
    lax.fori_loop(0, nf, group, 0)
    started_until = _NSLOT * nf + _NSLOT
    for c in range(_NSLOT * nf, _NCH2):
        b = c % _NSLOT
        if c >= started_until:
            start(c, b)
        finish(c, b)
    for b in range(_NSLOT):
        cb = max(c for c in range(_NCH2) if c % _NSLOT == b)
        prow = bufs[b][0]
        st = bufs[b][4]
        pltpu.make_async_copy(
            prow, g_hbm.at[pl.ds(base + cb * _C, _C)], st).wait()


@functools.partial(
    pl.kernel,
    mesh=_MESH,
    out_type=jax.ShapeDtypeStruct((_E2, _H), jnp.float32),
    scratch_types=(
        [pltpu.VMEM((_NCH2, _C), jnp.int32)] * 2
        + [pltpu.VMEM((_C, _H), jnp.float32)] * (2 * _NSLOT)
        + [pltpu.SemaphoreType.DMA] * (3 * _NSLOT)
    ),
)
def _sc_gather(p_hbm, q_hbm, dst3_hbm, src3_hbm, g_hbm, *rest):
    _sc_gather_body(p_hbm, q_hbm, dst3_hbm, src3_hbm, g_hbm, *rest)


def _sc_scatter_body(edge_base, h3_hbm, dst3_hbm, zeros_hbm, out_hbm,
                     idx_v, rows0, rows1, rows2, acc_sh, sr0, sr1, sr2):
    cid = lax.axis_index("c")
    sid = lax.axis_index("s")
    wid = sid * _NC + cid
    base = edge_base + wid * _PER_W

    pltpu.sync_copy(dst3_hbm.at[wid], idx_v)

    pltpu.sync_copy(zeros_hbm, acc_sh.at[pl.ds(sid * _RPS, _RPS)])
    plsc.subcore_barrier()

    bufs = ((rows0, sr0), (rows1, sr1), (rows2, sr2))

    def start(c, b):
        rows, sr = bufs[b]
        pltpu.async_copy(h3_hbm.at[pl.ds(base + c * _C, _C)], rows, sr)

    def wait_load(c, b):
        rows, sr = bufs[b]
        pltpu.make_async_copy(
            h3_hbm.at[pl.ds(base + c * _C, _C)], rows, sr).wait()

    def scat(c, b):
        rows, sr = bufs[b]
        return pltpu.async_copy(rows, acc_sh.at[idx_v.at[c]], sr, add=True)

    for b in range(3):
        start(b, b)

    def tri(k, carry):
        c0 = 3 * k
        hs = []
        for j in range(3):
            wait_load(c0 + j, j)
            hs.append(scat(c0 + j, j))
        for j in range(3):
            hs[j].wait()
            start(c0 + 3 + j, j)
        return carry

    nfull = (_NCH - 5) // 3
    lax.fori_loop(0, nfull, tri, 0)

    c0 = 3 * nfull
    hs = []
    for j in range(3):
        wait_load(c0 + j, j)
        hs.append(scat(c0 + j, j))
    for j in range(2):
        hs[j].wait()
        start(c0 + 3 + j, j)
    hs2 = []
    for j in range(2):
        wait_load(c0 + 3 + j, j)
        hs2.append(scat(c0 + 3 + j, j))
    hs[2].wait()
    hs2[0].wait()
    hs2[1].wait()
    plsc.subcore_barrier()

    pltpu.sync_copy(acc_sh.at[pl.ds(sid * _RPS, _RPS)],
                    out_hbm.at[cid, pl.ds(sid * _RPS, _RPS)])


def _make_scatter(edge_base):
    @functools.partial(
        pl.kernel,
        mesh=_MESH,
        out_type=jax.ShapeDtypeStruct((_NC, _NP, _H), jnp.float32),
        scratch_types=[
            pltpu.VMEM((_NCH, _C), jnp.int32),
            pltpu.VMEM((_C, _H), jnp.float32),
            pltpu.VMEM((_C, _H), jnp.float32),
            pltpu.VMEM((_C, _H), jnp.float32),
            pltpu.VMEM_SHARED((_NP, _H), jnp.float32),
            pltpu.SemaphoreType.DMA,
            pltpu.SemaphoreType.DMA,
            pltpu.SemaphoreType.DMA,
        ],
    )
    def _k(h3_hbm, dst3_hbm, zeros_hbm, *rest):
        _sc_scatter_body(edge_base, h3_hbm, dst3_hbm, zeros_hbm, *rest)

    return _k


_SC_SCATTER = (_make_scatter(0), _make_scatter(_E))


def kernel(x_rna, edge_index_rna, e_rna, x_atac, edge_index_atac, e_atac,
           x_cell, W1_rna, b1_rna, W2_rna, b2_rna, Wu_rna, bu_rna,
           W1_atac, b1_atac, W2_atac, b2_atac, Wu_atac, bu_atac, Wc, bc):
    src_r = edge_index_rna[0].astype(jnp.int32)
    dst_r = edge_index_rna[1].astype(jnp.int32)
    src_a = edge_index_atac[0].astype(jnp.int32)
    dst_a = edge_index_atac[1].astype(jnp.int32)

    x2 = jnp.concatenate([x_rna, x_atac], axis=0)
    wa2 = jnp.stack([W1_rna[:_D], W1_atac[:_D]])
    wb2 = jnp.stack([W1_rna[_D:2 * _D], W1_atac[_D:2 * _D]])
    p2, q2 = _node_tables(x2, wa2, wb2)
    src23 = jnp.concatenate([src_r, src_a + _N]).reshape(_NW, _NCH2, _C)
    dst23 = jnp.concatenate([dst_r, dst_a + _N]).reshape(_NW, _NCH2, _C)
    g2 = _sc_gather(p2, q2, dst23, src23)

    e2 = jnp.concatenate([e_rna, e_atac], axis=0)
    w1c2 = jnp.stack([W1_rna[2 * _D:], W1_atac[2 * _D:]])
    b12 = jnp.stack([b1_rna.reshape(1, _H), b1_atac.reshape(1, _H)])
    w22 = jnp.stack([W2_rna, W2_atac])
    b22 = jnp.stack([b2_rna.reshape(1, _H), b2_atac.reshape(1, _H)])
    h3 = _edge_mlp(g2, e2, w1c2, b12, w22, b22)

    zeros = jnp.zeros((_RPS, _H), jnp.float32)
    dst3_r = dst_r.reshape(_NW, _NCH, _C)
    dst3_a = dst_a.reshape(_NW, _NCH, _C)
    agg_r = _SC_SCATTER[0](h3, dst3_r, zeros)
    agg_a = _SC_SCATTER[1](h3, dst3_a, zeros)
    h_rna = _final_update(agg_r, Wu_rna, bu_rna)
    h_atac = _final_update(agg_a, Wu_atac, bu_atac)
    c = _cell_branch(x_cell, Wc, bc)
    return (h_rna, h_atac, c)


# speedup vs baseline: 1.1423x; 1.1423x over previous
import functools

import jax
import jax.numpy as jnp
from jax import lax
from jax.experimental import pallas as pl
from jax.experimental.pallas import tpu as pltpu
from jax.experimental.pallas import tpu_sc as plsc

_N = 10000
_E = 320000
_D = 128
_DE = 16
_H = 128

_NC = 2
_NS = 16
_NW = _NC * _NS
_C = 80
_PER_W = _E // _NW
_NCH = _PER_W // _C
_HW = _H // 2
_NP = 10240
_RPS = _NP // _NS
_ZR = 64


def _pq_body(x_ref, wa_ref, wb_ref, p_ref, q_ref):
    x = x_ref[...]
    p_ref[...] = jnp.dot(x, wa_ref[...], preferred_element_type=jnp.float32)
    q_ref[...] = jnp.dot(x, wb_ref[...], preferred_element_type=jnp.float32)


def _node_tables(x, wa, wb):
    bs = 2000
    return pl.pallas_call(
        _pq_body,
        grid=(_N // bs,),
        in_specs=[
            pl.BlockSpec((bs, _D), lambda i: (i, 0)),
            pl.BlockSpec((_D, _H), lambda i: (0, 0)),
            pl.BlockSpec((_D, _H), lambda i: (0, 0)),
        ],
        out_specs=[
            pl.BlockSpec((bs, _H), lambda i: (i, 0)),
            pl.BlockSpec((bs, _H), lambda i: (i, 0)),
        ],
        out_shape=[
            jax.ShapeDtypeStruct((_N, _H), jnp.float32),
            jax.ShapeDtypeStruct((_N, _H), jnp.float32),
        ],
    )(x, wa, wb)


def _edge_body(g_ref, e_ref, w1c_ref, b1_ref, w2_ref, b2_ref, h3_ref):
    pre = (g_ref[...]
           + jnp.dot(e_ref[...], w1c_ref[...],
                     preferred_element_type=jnp.float32) + b1_ref[...])
    h = jax.nn.silu(pre)
    h2 = jnp.dot(h.astype(jnp.bfloat16), w2_ref[...],
                 preferred_element_type=jnp.float32) + b2_ref[...]
    h2 = jax.nn.silu(h2)
    h3_ref[...] = jax.nn.silu(h2)


def _edge_mlp(g, e, w1c, b1, w2, b2):
    bs = 4000
    return pl.pallas_call(
        _edge_body,
        grid=(_E // bs,),
        in_specs=[
            pl.BlockSpec((bs, _H), lambda i: (i, 0)),
            pl.BlockSpec((bs, _DE), lambda i: (i, 0)),
            pl.BlockSpec((_DE, _H), lambda i: (0, 0)),
            pl.BlockSpec((1, _H), lambda i: (0, 0)),
            pl.BlockSpec((_H, _H), lambda i: (0, 0)),
            pl.BlockSpec((1, _H), lambda i: (0, 0)),
        ],
        out_specs=pl.BlockSpec((bs, _H), lambda i: (i, 0)),
        out_shape=jax.ShapeDtypeStruct((_E, _H), jnp.float32),
    )(g, e, w1c, b1.reshape(1, _H), w2.astype(jnp.bfloat16),
      b2.reshape(1, _H))


def _final_body(agg_ref, wu_ref, bu_ref, out_ref):
    a = agg_ref[0] + agg_ref[1]
    out_ref[...] = jnp.dot(a, wu_ref[...],
                           preferred_element_type=jnp.float32) + bu_ref[...]


def _final_update(agg, wu, bu):
    bs = 2000
    return pl.pallas_call(
        _final_body,
        grid=(_N // bs,),
        in_specs=[
            pl.BlockSpec((2, bs, _H), lambda i: (0, i, 0)),
            pl.BlockSpec((_H, _H), lambda i: (0, 0)),
            pl.BlockSpec((1, _H), lambda i: (0, 0)),
        ],
        out_specs=pl.BlockSpec((bs, _H), lambda i: (i, 0)),
        out_shape=jax.ShapeDtypeStruct((_N, _H), jnp.float32),
    )(agg, wu, bu.reshape(1, _H))


def _cell_body(x_ref, wc_ref, bc_ref, out_ref):
    out_ref[...] = jax.nn.silu(
        jnp.dot(x_ref[...], wc_ref[...], preferred_element_type=jnp.float32)
        + bc_ref[...])


def _cell_branch(x, wc, bc):
    bs = 2000
    return pl.pallas_call(
        _cell_body,
        grid=(_N // bs,),
        in_specs=[
            pl.BlockSpec((bs, _D), lambda i: (i, 0)),
            pl.BlockSpec((_D, _H), lambda i: (0, 0)),
            pl.BlockSpec((1, _H), lambda i: (0, 0)),
        ],
        out_specs=pl.BlockSpec((bs, _H), lambda i: (i, 0)),
        out_shape=jax.ShapeDtypeStruct((_N, _H), jnp.float32),
    )(x, wc, bc.reshape(1, _H))


_MESH = plsc.VectorSubcoreMesh(core_axis_name="c", subcore_axis_name="s")


_NSLOT = 4


def _sc_gather_body(p_hbm, q_hbm, dst3_hbm, src3_hbm, g_hbm,
                    idx_d, idx_s, *rest):
    wid = lax.axis_index("s") * _NC + lax.axis_index("c")
    base = wid * _PER_W

    pltpu.sync_copy(dst3_hbm.at[wid], idx_d)
    pltpu.sync_copy(src3_hbm.at[wid], idx_s)

    rows = rest[:2 * _NSLOT]
    sems = rest[2 * _NSLOT:]
    bufs = tuple(
        (rows[2 * b], rows[2 * b + 1],
         sems[3 * b], sems[3 * b + 1], sems[3 * b + 2])
        for b in range(_NSLOT))

    def start(c, b, first=False):
        prow, qrow, sp, sq, st = bufs[b]
        if not first:
            pltpu.make_async_copy(prow, g_hbm.at[pl.ds(base, _C)], st).wait()
        pltpu.async_copy(p_hbm.at[idx_d.at[c]], prow, sp)
        pltpu.async_copy(q_hbm.at[idx_s.at[c]], qrow, sq)

    def finish(c, b, last=False):
        prow, qrow, sp, sq, st = bufs[b]
        pltpu.make_async_copy(p_hbm.at[idx_d.at[c]], prow, sp).wait()
        pltpu.make_async_copy(q_hbm.at[idx_s.at[c]], qrow, sq).wait()

        def add_row(r, c2):
            for j in range(_H // 16):
                sl = pl.ds(j * 16, 16)
                plsc.addupdate(prow.at[r, sl], qrow[r, sl])
            return c2

        lax.fori_loop(0, _C, add_row, 0)
        dsub = pl.ds(base + c * _C, _C)
        pltpu.async_copy(prow, g_hbm.at[dsub], st)
        if last:
            pltpu.make_async_copy(prow, g_hbm.at[dsub], st).wait()

    for b in range(_NSLOT):
        start(b, b, first=True)

    def group(k, carry):
        c0 = _NSLOT * k
        for j in range(_NSLOT):
            finish(c0 + j, j)
            start(c0 + _NSLOT + j, j)
        return carry

    nf = _NCH // _NSLOT - 1
    lax.fori_loop(0, nf, group, 0)
    started_until = _NSLOT * nf + _NSLOT
    for c in range(_NSLOT * nf, _NCH):
        b = c % _NSLOT
        if c >= started_until:
            start(c, b)
        finish(c, b)
    for b in range(_NSLOT):
        cb = max(c for c in range(_NCH) if c % _NSLOT == b)
        prow = bufs[b][0]
        st = bufs[b][4]
        pltpu.make_async_copy(
            prow, g_hbm.at[pl.ds(base + cb * _C, _C)], st).wait()


@functools.partial(
    pl.kernel,
    mesh=_MESH,
    out_type=jax.ShapeDtypeStruct((_E, _H), jnp.float32),
    scratch_types=(
        [pltpu.VMEM((_NCH, _C), jnp.int32)] * 2
        + [pltpu.VMEM((_C, _H), jnp.float32)] * (2 * _NSLOT)
        + [pltpu.SemaphoreType.DMA] * (3 * _NSLOT)
    ),
)
def _sc_gather(p_hbm, q_hbm, dst3_hbm, src3_hbm, g_hbm, *rest):
    _sc_gather_body(p_hbm, q_hbm, dst3_hbm, src3_hbm, g_hbm, *rest)


def _sc_scatter_body(edge_base, h3_hbm, dst3_hbm, zeros_hbm, out_hbm,
                     idx_v, rows0, rows1, rows2, acc_sh, sr0, sr1, sr2):
    cid = lax.axis_index("c")
    sid = lax.axis_index("s")
    wid = sid * _NC + cid
    base = edge_base + wid * _PER_W

    pltpu.sync_copy(dst3_hbm.at[wid], idx_v)

    pltpu.sync_copy(zeros_hbm, acc_sh.at[pl.ds(sid * _RPS, _RPS)])
    plsc.subcore_barrier()

    bufs = ((rows0, sr0), (rows1, sr1), (rows2, sr2))

    def start(c, b):
        rows, sr = bufs[b]
        pltpu.async_copy(h3_hbm.at[pl.ds(base + c * _C, _C)], rows, sr)

    def wait_load(c, b):
        rows, sr = bufs[b]
        pltpu.make_async_copy(
            h3_hbm.at[pl.ds(base + c * _C, _C)], rows, sr).wait()

    def scat(c, b):
        rows, sr = bufs[b]
        return pltpu.async_copy(rows, acc_sh.at[idx_v.at[c]], sr, add=True)

    for b in range(3):
        start(b, b)

    def tri(k, carry):
        c0 = 3 * k
        hs = []
        for j in range(3):
            wait_load(c0 + j, j)
            hs.append(scat(c0 + j, j))
        for j in range(3):
            hs[j].wait()
            start(c0 + 3 + j, j)
        return carry

    nfull = (_NCH - 5) // 3
    lax.fori_loop(0, nfull, tri, 0)

    c0 = 3 * nfull
    hs = []
    for j in range(3):
        wait_load(c0 + j, j)
        hs.append(scat(c0 + j, j))
    for j in range(2):
        hs[j].wait()
        start(c0 + 3 + j, j)
    hs2 = []
    for j in range(2):
        wait_load(c0 + 3 + j, j)
        hs2.append(scat(c0 + 3 + j, j))
    hs[2].wait()
    hs2[0].wait()
    hs2[1].wait()
    plsc.subcore_barrier()

    pltpu.sync_copy(acc_sh.at[pl.ds(sid * _RPS, _RPS)],
                    out_hbm.at[cid, pl.ds(sid * _RPS, _RPS)])


@functools.partial(
    pl.kernel,
    mesh=_MESH,
    out_type=jax.ShapeDtypeStruct((_NC, _NP, _H), jnp.float32),
    scratch_types=[
        pltpu.VMEM((_NCH, _C), jnp.int32),
        pltpu.VMEM((_C, _H), jnp.float32),
        pltpu.VMEM((_C, _H), jnp.float32),
        pltpu.VMEM((_C, _H), jnp.float32),
        pltpu.VMEM_SHARED((_NP, _H), jnp.float32),
        pltpu.SemaphoreType.DMA,
        pltpu.SemaphoreType.DMA,
        pltpu.SemaphoreType.DMA,
    ],
)
def _sc_scatter(h3_hbm, dst3_hbm, zeros_hbm, *rest):
    _sc_scatter_body(0, h3_hbm, dst3_hbm, zeros_hbm, *rest)


def _modality(x, edge_index, e, w1, b1, w2, b2, wu, bu):
    src3 = edge_index[0].astype(jnp.int32).reshape(_NW, _NCH, _C)
    dst3 = edge_index[1].astype(jnp.int32).reshape(_NW, _NCH, _C)
    p, q = _node_tables(x, w1[:_D], w1[_D:2 * _D])
    g = _sc_gather(p, q, dst3, src3)
    h3 = _edge_mlp(g, e, w1[2 * _D:], b1, w2, b2)
    zeros = jnp.zeros((_RPS, _H), jnp.float32)
    agg = _sc_scatter(h3, dst3, zeros)
    return _final_update(agg, wu, bu)


def kernel(x_rna, edge_index_rna, e_rna, x_atac, edge_index_atac, e_atac,
           x_cell, W1_rna, b1_rna, W2_rna, b2_rna, Wu_rna, bu_rna,
           W1_atac, b1_atac, W2_atac, b2_atac, Wu_atac, bu_atac, Wc, bc):
    h_rna = _modality(x_rna, edge_index_rna, e_rna,
                      W1_rna, b1_rna, W2_rna, b2_rna, Wu_rna, bu_rna)
    h_atac = _modality(x_atac, edge_index_atac, e_atac,
                       W1_atac, b1_atac, W2_atac, b2_atac, Wu_atac, bu_atac)
    c = _cell_branch(x_cell, Wc, bc)
    return (h_rna, h_atac, c)


# speedup vs baseline: 1.1670x; 1.0217x over previous
import functools

import jax
import jax.numpy as jnp
from jax import lax
from jax.experimental import pallas as pl
from jax.experimental.pallas import tpu as pltpu
from jax.experimental.pallas import tpu_sc as plsc

_N = 10000
_E = 320000
_D = 128
_DE = 16
_H = 128

_NC = 2
_NS = 16
_NW = _NC * _NS
_C = 80
_PER_W = _E // _NW
_NCH = _PER_W // _C
_HW = _H // 2
_NP = 10240
_RPS = _NP // _NS
_ZR = 64


def _pq_body(x_ref, wa_ref, wb_ref, p_ref, q_ref):
    x = x_ref[...]
    p_ref[...] = jnp.dot(x, wa_ref[...], preferred_element_type=jnp.float32)
    q_ref[...] = jnp.dot(x, wb_ref[...], preferred_element_type=jnp.float32)


def _node_tables(x, wa, wb):
    bs = 2000
    return pl.pallas_call(
        _pq_body,
        grid=(_N // bs,),
        in_specs=[
            pl.BlockSpec((bs, _D), lambda i: (i, 0)),
            pl.BlockSpec((_D, _H), lambda i: (0, 0)),
            pl.BlockSpec((_D, _H), lambda i: (0, 0)),
        ],
        out_specs=[
            pl.BlockSpec((bs, _H), lambda i: (i, 0)),
            pl.BlockSpec((bs, _H), lambda i: (i, 0)),
        ],
        out_shape=[
            jax.ShapeDtypeStruct((_N, _H), jnp.float32),
            jax.ShapeDtypeStruct((_N, _H), jnp.float32),
        ],
    )(x, wa, wb)


def _edge_body(g_ref, e_ref, w1c_ref, b1_ref, w2_ref, b2_ref, h3_ref):
    pre = (g_ref[...]
           + jnp.dot(e_ref[...], w1c_ref[...],
                     preferred_element_type=jnp.float32) + b1_ref[...])
    h = jax.nn.silu(pre)
    h2 = jnp.dot(h.astype(jnp.bfloat16), w2_ref[...],
                 preferred_element_type=jnp.float32) + b2_ref[...]
    h2 = jax.nn.silu(h2)
    h3_ref[...] = jax.nn.silu(h2)


def _edge_mlp(g, e, w1c, b1, w2, b2):
    bs = 8000
    return pl.pallas_call(
        _edge_body,
        grid=(_E // bs,),
        in_specs=[
            pl.BlockSpec((bs, _H), lambda i: (i, 0)),
            pl.BlockSpec((bs, _DE), lambda i: (i, 0)),
            pl.BlockSpec((_DE, _H), lambda i: (0, 0)),
            pl.BlockSpec((1, _H), lambda i: (0, 0)),
            pl.BlockSpec((_H, _H), lambda i: (0, 0)),
            pl.BlockSpec((1, _H), lambda i: (0, 0)),
        ],
        out_specs=pl.BlockSpec((bs, _H), lambda i: (i, 0)),
        out_shape=jax.ShapeDtypeStruct((_E, _H), jnp.float32),
    )(g, e, w1c, b1.reshape(1, _H), w2.astype(jnp.bfloat16),
      b2.reshape(1, _H))


def _final_body(agg_ref, wu_ref, bu_ref, out_ref):
    a = agg_ref[0] + agg_ref[1]
    out_ref[...] = jnp.dot(a, wu_ref[...],
                           preferred_element_type=jnp.float32) + bu_ref[...]


def _final_update(agg, wu, bu):
    bs = 2000
    return pl.pallas_call(
        _final_body,
        grid=(_N // bs,),
        in_specs=[
            pl.BlockSpec((2, bs, _H), lambda i: (0, i, 0)),
            pl.BlockSpec((_H, _H), lambda i: (0, 0)),
            pl.BlockSpec((1, _H), lambda i: (0, 0)),
        ],
        out_specs=pl.BlockSpec((bs, _H), lambda i: (i, 0)),
        out_shape=jax.ShapeDtypeStruct((_N, _H), jnp.float32),
    )(agg, wu, bu.reshape(1, _H))


def _cell_body(x_ref, wc_ref, bc_ref, out_ref):
    out_ref[...] = jax.nn.silu(
        jnp.dot(x_ref[...], wc_ref[...], preferred_element_type=jnp.float32)
        + bc_ref[...])


def _cell_branch(x, wc, bc):
    bs = 2000
    return pl.pallas_call(
        _cell_body,
        grid=(_N // bs,),
        in_specs=[
            pl.BlockSpec((bs, _D), lambda i: (i, 0)),
            pl.BlockSpec((_D, _H), lambda i: (0, 0)),
            pl.BlockSpec((1, _H), lambda i: (0, 0)),
        ],
        out_specs=pl.BlockSpec((bs, _H), lambda i: (i, 0)),
        out_shape=jax.ShapeDtypeStruct((_N, _H), jnp.float32),
    )(x, wc, bc.reshape(1, _H))


_MESH = plsc.VectorSubcoreMesh(core_axis_name="c", subcore_axis_name="s")


_NSLOT = 4


def _sc_gather_body(p_hbm, q_hbm, dst3_hbm, src3_hbm, g_hbm,
                    idx_d, idx_s, *rest):
    wid = lax.axis_index("s") * _NC + lax.axis_index("c")
    base = wid * _PER_W

    pltpu.sync_copy(dst3_hbm.at[wid], idx_d)
    pltpu.sync_copy(src3_hbm.at[wid], idx_s)

    rows = rest[:2 * _NSLOT]
    sems = rest[2 * _NSLOT:]
    bufs = tuple(
        (rows[2 * b], rows[2 * b + 1],
         sems[3 * b], sems[3 * b + 1], sems[3 * b + 2])
        for b in range(_NSLOT))

    def start(c, b, first=False):
        prow, qrow, sp, sq, st = bufs[b]
        if not first:
            pltpu.make_async_copy(prow, g_hbm.at[pl.ds(base, _C)], st).wait()
        pltpu.async_copy(p_hbm.at[idx_d.at[c]], prow, sp)
        pltpu.async_copy(q_hbm.at[idx_s.at[c]], qrow, sq)

    def finish(c, b, last=False):
        prow, qrow, sp, sq, st = bufs[b]
        pltpu.make_async_copy(p_hbm.at[idx_d.at[c]], prow, sp).wait()
        pltpu.make_async_copy(q_hbm.at[idx_s.at[c]], qrow, sq).wait()

        def add_row(r, c2):
            for j in range(_H // 16):
                sl = pl.ds(j * 16, 16)
                plsc.addupdate(prow.at[r, sl], qrow[r, sl])
            return c2

        lax.fori_loop(0, _C, add_row, 0)
        dsub = pl.ds(base + c * _C, _C)
        pltpu.async_copy(prow, g_hbm.at[dsub], st)
        if last:
            pltpu.make_async_copy(prow, g_hbm.at[dsub], st).wait()

    for b in range(_NSLOT):
        start(b, b, first=True)

    def group(k, carry):
        c0 = _NSLOT * k
        for j in range(_NSLOT):
            finish(c0 + j, j)
            start(c0 + _NSLOT + j, j)
        return carry

    nf = _NCH // _NSLOT - 1
    lax.fori_loop(0, nf, group, 0)
    started_until = _NSLOT * nf + _NSLOT
    for c in range(_NSLOT * nf, _NCH):
        b = c % _NSLOT
        if c >= started_until:
            start(c, b)
        finish(c, b)
    for b in range(_NSLOT):
        cb = max(c for c in range(_NCH) if c % _NSLOT == b)
        prow = bufs[b][0]
        st = bufs[b][4]
        pltpu.make_async_copy(
            prow, g_hbm.at[pl.ds(base + cb * _C, _C)], st).wait()


@functools.partial(
    pl.kernel,
    mesh=_MESH,
    out_type=jax.ShapeDtypeStruct((_E, _H), jnp.float32),
    scratch_types=(
        [pltpu.VMEM((_NCH, _C), jnp.int32)] * 2
        + [pltpu.VMEM((_C, _H), jnp.float32)] * (2 * _NSLOT)
        + [pltpu.SemaphoreType.DMA] * (3 * _NSLOT)
    ),
)
def _sc_gather(p_hbm, q_hbm, dst3_hbm, src3_hbm, g_hbm, *rest):
    _sc_gather_body(p_hbm, q_hbm, dst3_hbm, src3_hbm, g_hbm, *rest)


def _sc_scatter_body(edge_base, h3_hbm, dst3_hbm, zeros_hbm, out_hbm,
                     idx_v, rows0, rows1, rows2, acc_sh, sr0, sr1, sr2):
    cid = lax.axis_index("c")
    sid = lax.axis_index("s")
    wid = sid * _NC + cid
    base = edge_base + wid * _PER_W

    pltpu.sync_copy(dst3_hbm.at[wid], idx_v)

    pltpu.sync_copy(zeros_hbm, acc_sh.at[pl.ds(sid * _RPS, _RPS)])
    plsc.subcore_barrier()

    bufs = ((rows0, sr0), (rows1, sr1), (rows2, sr2))

    def start(c, b):
        rows, sr = bufs[b]
        pltpu.async_copy(h3_hbm.at[pl.ds(base + c * _C, _C)], rows, sr)

    def wait_load(c, b):
        rows, sr = bufs[b]
        pltpu.make_async_copy(
            h3_hbm.at[pl.ds(base + c * _C, _C)], rows, sr).wait()

    def scat(c, b):
        rows, sr = bufs[b]
        return pltpu.async_copy(rows, acc_sh.at[idx_v.at[c]], sr, add=True)

    for b in range(3):
        start(b, b)

    def tri(k, carry):
        c0 = 3 * k
        hs = []
        for j in range(3):
            wait_load(c0 + j, j)
            hs.append(scat(c0 + j, j))
        for j in range(3):
            hs[j].wait()
            start(c0 + 3 + j, j)
        return carry

    nfull = (_NCH - 5) // 3
    lax.fori_loop(0, nfull, tri, 0)

    c0 = 3 * nfull
    hs = []
    for j in range(3):
        wait_load(c0 + j, j)
        hs.append(scat(c0 + j, j))
    for j in range(2):
        hs[j].wait()
        start(c0 + 3 + j, j)
    hs2 = []
    for j in range(2):
        wait_load(c0 + 3 + j, j)
        hs2.append(scat(c0 + 3 + j, j))
    hs[2].wait()
    hs2[0].wait()
    hs2[1].wait()
    plsc.subcore_barrier()

    pltpu.sync_copy(acc_sh.at[pl.ds(sid * _RPS, _RPS)],
                    out_hbm.at[cid, pl.ds(sid * _RPS, _RPS)])


@functools.partial(
    pl.kernel,
    mesh=_MESH,
    out_type=jax.ShapeDtypeStruct((_NC, _NP, _H), jnp.float32),
    scratch_types=[
        pltpu.VMEM((_NCH, _C), jnp.int32),
        pltpu.VMEM((_C, _H), jnp.float32),
        pltpu.VMEM((_C, _H), jnp.float32),
        pltpu.VMEM((_C, _H), jnp.float32),
        pltpu.VMEM_SHARED((_NP, _H), jnp.float32),
        pltpu.SemaphoreType.DMA,
        pltpu.SemaphoreType.DMA,
        pltpu.SemaphoreType.DMA,
    ],
)
def _sc_scatter(h3_hbm, dst3_hbm, zeros_hbm, *rest):
    _sc_scatter_body(0, h3_hbm, dst3_hbm, zeros_hbm, *rest)


def _modality(x, edge_index, e, w1, b1, w2, b2, wu, bu):
    src3 = edge_index[0].astype(jnp.int32).reshape(_NW, _NCH, _C)
    dst3 = edge_index[1].astype(jnp.int32).reshape(_NW, _NCH, _C)
    p, q = _node_tables(x, w1[:_D], w1[_D:2 * _D])
    g = _sc_gather(p, q, dst3, src3)
    h3 = _edge_mlp(g, e, w1[2 * _D:], b1, w2, b2)
    zeros = jnp.zeros((_RPS, _H), jnp.float32)
    agg = _sc_scatter(h3, dst3, zeros)
    return _final_update(agg, wu, bu)


def kernel(x_rna, edge_index_rna, e_rna, x_atac, edge_index_atac, e_atac,
           x_cell, W1_rna, b1_rna, W2_rna, b2_rna, Wu_rna, bu_rna,
           W1_atac, b1_atac, W2_atac, b2_atac, Wu_atac, bu_atac, Wc, bc):
    h_rna = _modality(x_rna, edge_index_rna, e_rna,
                      W1_rna, b1_rna, W2_rna, b2_rna, Wu_rna, bu_rna)
    h_atac = _modality(x_atac, edge_index_atac, e_atac,
                       W1_atac, b1_atac, W2_atac, b2_atac, Wu_atac, bu_atac)
    c = _cell_branch(x_cell, Wc, bc)
    return (h_rna, h_atac, c)


# speedup vs baseline: 1.1796x; 1.0108x over previous
import functools

import jax
import jax.numpy as jnp
from jax import lax
from jax.experimental import pallas as pl
from jax.experimental.pallas import tpu as pltpu
from jax.experimental.pallas import tpu_sc as plsc

_N = 10000
_E = 320000
_D = 128
_DE = 16
_H = 128

_NC = 2
_NS = 16
_NW = _NC * _NS
_C = 80
_PER_W = _E // _NW
_NCH = _PER_W // _C
_HW = _H // 2
_NP = 10240
_RPS = _NP // _NS
_ZR = 64


def _pq_body(x_ref, wa_ref, wb_ref, p_ref, q_ref):
    x = x_ref[...]
    p_ref[...] = jnp.dot(x, wa_ref[...], preferred_element_type=jnp.float32)
    q_ref[...] = jnp.dot(x, wb_ref[...], preferred_element_type=jnp.float32)


def _node_tables(x, wa, wb):
    bs = 2000
    return pl.pallas_call(
        _pq_body,
        grid=(_N // bs,),
        in_specs=[
            pl.BlockSpec((bs, _D), lambda i: (i, 0)),
            pl.BlockSpec((_D, _H), lambda i: (0, 0)),
            pl.BlockSpec((_D, _H), lambda i: (0, 0)),
        ],
        out_specs=[
            pl.BlockSpec((bs, _H), lambda i: (i, 0)),
            pl.BlockSpec((bs, _H), lambda i: (i, 0)),
        ],
        out_shape=[
            jax.ShapeDtypeStruct((_N, _H), jnp.float32),
            jax.ShapeDtypeStruct((_N, _H), jnp.float32),
        ],
    )(x, wa, wb)


def _edge_body(g_ref, e_ref, w1c_ref, b1_ref, w2_ref, b2_ref, h3_ref):
    pre = (g_ref[...]
           + jnp.dot(e_ref[...], w1c_ref[...],
                     preferred_element_type=jnp.float32) + b1_ref[...])
    h = jax.nn.silu(pre)
    h2 = jnp.dot(h.astype(jnp.bfloat16), w2_ref[...],
                 preferred_element_type=jnp.float32) + b2_ref[...]
    h2 = jax.nn.silu(h2)
    h3_ref[...] = jax.nn.silu(h2)


def _edge_mlp(g, e, w1c, b1, w2, b2):
    bs = 16000
    return pl.pallas_call(
        _edge_body,
        grid=(_E // bs,),
        in_specs=[
            pl.BlockSpec((bs, _H), lambda i: (i, 0)),
            pl.BlockSpec((bs, _DE), lambda i: (i, 0)),
            pl.BlockSpec((_DE, _H), lambda i: (0, 0)),
            pl.BlockSpec((1, _H), lambda i: (0, 0)),
            pl.BlockSpec((_H, _H), lambda i: (0, 0)),
            pl.BlockSpec((1, _H), lambda i: (0, 0)),
        ],
        out_specs=pl.BlockSpec((bs, _H), lambda i: (i, 0)),
        out_shape=jax.ShapeDtypeStruct((_E, _H), jnp.float32),
    )(g, e, w1c, b1.reshape(1, _H), w2.astype(jnp.bfloat16),
      b2.reshape(1, _H))


def _final_body(agg_ref, wu_ref, bu_ref, out_ref):
    a = agg_ref[0] + agg_ref[1]
    out_ref[...] = jnp.dot(a, wu_ref[...],
                           preferred_element_type=jnp.float32) + bu_ref[...]


def _final_update(agg, wu, bu):
    bs = 2000
    return pl.pallas_call(
        _final_body,
        grid=(_N // bs,),
        in_specs=[
            pl.BlockSpec((2, bs, _H), lambda i: (0, i, 0)),
            pl.BlockSpec((_H, _H), lambda i: (0, 0)),
            pl.BlockSpec((1, _H), lambda i: (0, 0)),
        ],
        out_specs=pl.BlockSpec((bs, _H), lambda i: (i, 0)),
        out_shape=jax.ShapeDtypeStruct((_N, _H), jnp.float32),
    )(agg, wu, bu.reshape(1, _H))


def _cell_body(x_ref, wc_ref, bc_ref, out_ref):
    out_ref[...] = jax.nn.silu(
        jnp.dot(x_ref[...], wc_ref[...], preferred_element_type=jnp.float32)
        + bc_ref[...])


def _cell_branch(x, wc, bc):
    bs = 2000
    return pl.pallas_call(
        _cell_body,
        grid=(_N // bs,),
        in_specs=[
            pl.BlockSpec((bs, _D), lambda i: (i, 0)),
            pl.BlockSpec((_D, _H), lambda i: (0, 0)),
            pl.BlockSpec((1, _H), lambda i: (0, 0)),
        ],
        out_specs=pl.BlockSpec((bs, _H), lambda i: (i, 0)),
        out_shape=jax.ShapeDtypeStruct((_N, _H), jnp.float32),
    )(x, wc, bc.reshape(1, _H))


_MESH = plsc.VectorSubcoreMesh(core_axis_name="c", subcore_axis_name="s")


_NSLOT = 4


def _sc_gather_body(p_hbm, q_hbm, dst3_hbm, src3_hbm, g_hbm,
                    idx_d, idx_s, *rest):
    wid = lax.axis_index("s") * _NC + lax.axis_index("c")
    base = wid * _PER_W

    pltpu.sync_copy(dst3_hbm.at[wid], idx_d)
    pltpu.sync_copy(src3_hbm.at[wid], idx_s)

    rows = rest[:2 * _NSLOT]
    sems = rest[2 * _NSLOT:]
    bufs = tuple(
        (rows[2 * b], rows[2 * b + 1],
         sems[3 * b], sems[3 * b + 1], sems[3 * b + 2])
        for b in range(_NSLOT))

    def start(c, b, first=False):
        prow, qrow, sp, sq, st = bufs[b]
        if not first:
            pltpu.make_async_copy(prow, g_hbm.at[pl.ds(base, _C)], st).wait()
        pltpu.async_copy(p_hbm.at[idx_d.at[c]], prow, sp)
        pltpu.async_copy(q_hbm.at[idx_s.at[c]], qrow, sq)

    def finish(c, b, last=False):
        prow, qrow, sp, sq, st = bufs[b]
        pltpu.make_async_copy(p_hbm.at[idx_d.at[c]], prow, sp).wait()
        pltpu.make_async_copy(q_hbm.at[idx_s.at[c]], qrow, sq).wait()

        def add_row(r, c2):
            for j in range(_H // 16):
                sl = pl.ds(j * 16, 16)
                plsc.addupdate(prow.at[r, sl], qrow[r, sl])
            return c2

        lax.fori_loop(0, _C, add_row, 0)
        dsub = pl.ds(base + c * _C, _C)
        pltpu.async_copy(prow, g_hbm.at[dsub], st)
        if last:
            pltpu.make_async_copy(prow, g_hbm.at[dsub], st).wait()

    for b in range(_NSLOT):
        start(b, b, first=True)

    def group(k, carry):
        c0 = _NSLOT * k
        for j in range(_NSLOT):
            finish(c0 + j, j)
            start(c0 + _NSLOT + j, j)
        return carry

    nf = _NCH // _NSLOT - 1
    lax.fori_loop(0, nf, group, 0)
    started_until = _NSLOT * nf + _NSLOT
    for c in range(_NSLOT * nf, _NCH):
        b = c % _NSLOT
        if c >= started_until:
            start(c, b)
        finish(c, b)
    for b in range(_NSLOT):
        cb = max(c for c in range(_NCH) if c % _NSLOT == b)
        prow = bufs[b][0]
        st = bufs[b][4]
        pltpu.make_async_copy(
            prow, g_hbm.at[pl.ds(base + cb * _C, _C)], st).wait()


@functools.partial(
    pl.kernel,
    mesh=_MESH,
    out_type=jax.ShapeDtypeStruct((_E, _H), jnp.float32),
    scratch_types=(
        [pltpu.VMEM((_NCH, _C), jnp.int32)] * 2
        + [pltpu.VMEM((_C, _H), jnp.float32)] * (2 * _NSLOT)
        + [pltpu.SemaphoreType.DMA] * (3 * _NSLOT)
    ),
)
def _sc_gather(p_hbm, q_hbm, dst3_hbm, src3_hbm, g_hbm, *rest):
    _sc_gather_body(p_hbm, q_hbm, dst3_hbm, src3_hbm, g_hbm, *rest)


def _sc_scatter_body(edge_base, h3_hbm, dst3_hbm, zeros_hbm, out_hbm,
                     idx_v, rows0, rows1, rows2, acc_sh, sr0, sr1, sr2):
    cid = lax.axis_index("c")
    sid = lax.axis_index("s")
    wid = sid * _NC + cid
    base = edge_base + wid * _PER_W

    pltpu.sync_copy(dst3_hbm.at[wid], idx_v)

    pltpu.sync_copy(zeros_hbm, acc_sh.at[pl.ds(sid * _RPS, _RPS)])
    plsc.subcore_barrier()

    bufs = ((rows0, sr0), (rows1, sr1), (rows2, sr2))

    def start(c, b):
        rows, sr = bufs[b]
        pltpu.async_copy(h3_hbm.at[pl.ds(base + c * _C, _C)], rows, sr)

    def wait_load(c, b):
        rows, sr = bufs[b]
        pltpu.make_async_copy(
            h3_hbm.at[pl.ds(base + c * _C, _C)], rows, sr).wait()

    def scat(c, b):
        rows, sr = bufs[b]
        return pltpu.async_copy(rows, acc_sh.at[idx_v.at[c]], sr, add=True)

    for b in range(3):
        start(b, b)

    def tri(k, carry):
        c0 = 3 * k
        hs = []
        for j in range(3):
            wait_load(c0 + j, j)
            hs.append(scat(c0 + j, j))
        for j in range(3):
            hs[j].wait()
            start(c0 + 3 + j, j)
        return carry

    nfull = (_NCH - 5) // 3
    lax.fori_loop(0, nfull, tri, 0)

    c0 = 3 * nfull
    hs = []
    for j in range(3):
        wait_load(c0 + j, j)
        hs.append(scat(c0 + j, j))
    for j in range(2):
        hs[j].wait()
        start(c0 + 3 + j, j)
    hs2 = []
    for j in range(2):
        wait_load(c0 + 3 + j, j)
        hs2.append(scat(c0 + 3 + j, j))
    hs[2].wait()
    hs2[0].wait()
    hs2[1].wait()
    plsc.subcore_barrier()

    pltpu.sync_copy(acc_sh.at[pl.ds(sid * _RPS, _RPS)],
                    out_hbm.at[cid, pl.ds(sid * _RPS, _RPS)])


@functools.partial(
    pl.kernel,
    mesh=_MESH,
    out_type=jax.ShapeDtypeStruct((_NC, _NP, _H), jnp.float32),
    scratch_types=[
        pltpu.VMEM((_NCH, _C), jnp.int32),
        pltpu.VMEM((_C, _H), jnp.float32),
        pltpu.VMEM((_C, _H), jnp.float32),
        pltpu.VMEM((_C, _H), jnp.float32),
        pltpu.VMEM_SHARED((_NP, _H), jnp.float32),
        pltpu.SemaphoreType.DMA,
        pltpu.SemaphoreType.DMA,
        pltpu.SemaphoreType.DMA,
    ],
)
def _sc_scatter(h3_hbm, dst3_hbm, zeros_hbm, *rest):
    _sc_scatter_body(0, h3_hbm, dst3_hbm, zeros_hbm, *rest)


def _modality(x, edge_index, e, w1, b1, w2, b2, wu, bu):
    src3 = edge_index[0].astype(jnp.int32).reshape(_NW, _NCH, _C)
    dst3 = edge_index[1].astype(jnp.int32).reshape(_NW, _NCH, _C)
    p, q = _node_tables(x, w1[:_D], w1[_D:2 * _D])
    g = _sc_gather(p, q, dst3, src3)
    h3 = _edge_mlp(g, e, w1[2 * _D:], b1, w2, b2)
    zeros = jnp.zeros((_RPS, _H), jnp.float32)
    agg = _sc_scatter(h3, dst3, zeros)
    return _final_update(agg, wu, bu)


def kernel(x_rna, edge_index_rna, e_rna, x_atac, edge_index_atac, e_atac,
           x_cell, W1_rna, b1_rna, W2_rna, b2_rna, Wu_rna, bu_rna,
           W1_atac, b1_atac, W2_atac, b2_atac, Wu_atac, bu_atac, Wc, bc):
    h_rna = _modality(x_rna, edge_index_rna, e_rna,
                      W1_rna, b1_rna, W2_rna, b2_rna, Wu_rna, bu_rna)
    h_atac = _modality(x_atac, edge_index_atac, e_atac,
                       W1_atac, b1_atac, W2_atac, b2_atac, Wu_atac, bu_atac)
    c = _cell_branch(x_cell, Wc, bc)
    return (h_rna, h_atac, c)
